# baseline (speedup 1.0000x reference)
import functools

import jax
import jax.numpy as jnp
from jax import lax
from jax.experimental import pallas as pl
from jax.experimental.pallas import tpu as pltpu
from jax.experimental.pallas import tpu_sc as plsc

NC = 2
NS = 16
NW = NC * NS

C = 512


def _body(idx_hbm, w_hbm, s_hbm, m_hbm, out_hbm,
          idx_v, rows_v, s_v, m_v, o_v, sem_w, sem_s, sem_m, *, rpw, V):
  wid = lax.axis_index("s") * NC + lax.axis_index("c")
  base0 = wid * rpw
  w32 = w_hbm
  half = lax.iota(jnp.int32, 16) >> 1
  half8 = half + 8
  even_lane = (lax.iota(jnp.int32, 16) & 1) == 0

  def chunk_body(ci, _):
    base = pl.multiple_of(base0 + ci * C, C)
    pltpu.sync_copy(idx_hbm.at[pl.ds(base, C)], idx_v)
    cw = pltpu.async_copy(w32.at[idx_v], rows_v, sem_w)
    cs = pltpu.async_copy(s_hbm.at[idx_v], s_v, sem_s)
    cm = pltpu.async_copy(m_hbm.at[idx_v], m_v, sem_m)
    cw.wait()
    cs.wait()
    cm.wait()

    def group_body(g, _):
      gb = g * 16
      s16 = s_v[pl.ds(gb, 16)]
      m16 = m_v[pl.ds(gb, 16)]
      t16 = s16 * m16
      for i in range(16):
        vs = lax.broadcast_in_dim(s16[i], (16,), ())
        vt = lax.broadcast_in_dim(t16[i], (16,), ())
        w = rows_v[gb + i]
        b0 = (w << 24) >> 24
        b1 = (w << 16) >> 24
        b2 = (w << 8) >> 24
        b3 = w >> 24
        y0 = b0.astype(jnp.float32) * vs + vt
        y1 = b1.astype(jnp.float32) * vs + vt
        y2 = b2.astype(jnp.float32) * vs + vt
        y3 = b3.astype(jnp.float32) * vs + vt
        zlo = plsc.bitcast(
            plsc.pack(y0, y1, format=plsc.PackFormat.INTERLEAVED), jnp.int32)
        zhi = plsc.bitcast(
            plsc.pack(y2, y3, format=plsc.PackFormat.INTERLEAVED), jnp.int32)
        wa = jnp.where(even_lane,
                       jnp.take_along_axis(zlo, half, axis=0),
                       jnp.take_along_axis(zhi, half, axis=0))
        wb_ = jnp.where(even_lane,
                        jnp.take_along_axis(zlo, half8, axis=0),
                        jnp.take_along_axis(zhi, half8, axis=0))
        o_v[gb + i, pl.ds(0, 32)] = plsc.bitcast(wa, jnp.bfloat16)
        o_v[gb + i, pl.ds(32, 32)] = plsc.bitcast(wb_, jnp.bfloat16)
      return ()

    lax.fori_loop(0, C // 16, group_body, (), unroll=False)
    pltpu.sync_copy(o_v, out_hbm.at[pl.ds(base, C)])
    return ()

  lax.fori_loop(0, rpw // C, chunk_body, (), unroll=False)


def kernel(idx, weight, scales, means):
  B, L = idx.shape
  V, D = weight.shape
  BL = B * L
  rpw = BL // NW

  idxf = idx.reshape(BL)
  wb = jax.lax.bitcast_convert_type(
      weight.reshape(V, D // 4, 4), jnp.int32)
  sf = scales.reshape(V)
  mf = means.reshape(V)

  mesh = plsc.VectorSubcoreMesh(core_axis_name="c", subcore_axis_name="s")
  out = pl.kernel(
      functools.partial(_body, rpw=rpw, V=V),
      out_type=jax.ShapeDtypeStruct((BL, D), jnp.bfloat16),
      mesh=mesh,
      compiler_params=pltpu.CompilerParams(
          needs_layout_passes=False, use_tc_tiling_on_sc=False),
      scratch_types=[
          pltpu.VMEM((C,), jnp.int32),
          pltpu.VMEM((C, 16), jnp.int32),
          pltpu.VMEM((C,), jnp.float32),
          pltpu.VMEM((C,), jnp.float32),
          pltpu.VMEM((C, D), jnp.bfloat16),
          pltpu.SemaphoreType.DMA,
          pltpu.SemaphoreType.DMA,
          pltpu.SemaphoreType.DMA,
      ],
  )(idxf, wb, sf, mf)
  return out.reshape(B, L, D)
